# baseline (speedup 1.0000x reference)
import jax
import jax.numpy as jnp
from jax import lax
from jax.experimental import pallas as pl
from jax.experimental.pallas import tpu as pltpu

EPS = 0.1
THRESH = 0.1
MAX_ITER = 30
MAX_NUM = 3000
K = 8
W = 224

N = 3072
B = 256
CB = 512
NRB = N // B
NCB = N // CB
BIG = 1e9


def _gw_body(xs_ref, ys_ref, xt_ref, yt_ref, vs_ref, vt_ref, mu_ref, nu_ref,
             out_ref, ebuf, ebf, nrm_s, rv_s, rj_s, cv_s, ci_s, u_s, v_s, av_s, bv_s):
    b = pl.program_id(0)

    @pl.when(b == 0)
    def _():
        out_ref[0, 0] = 0.0

    vs = vs_ref[0, 0, :]
    vt = vt_ref[0, 0, :]
    nsi = jnp.sum(vs).astype(jnp.int32)
    nti = jnp.sum(vt).astype(jnp.int32)
    nbs = jnp.maximum((nsi + B - 1) // B, 1)
    nbt2 = jnp.maximum((nti + CB - 1) // CB, 1)
    inf = jnp.float32(jnp.inf)

    def row_block(bi, _):
        r0 = bi * B
        xi = xs_ref[0, 0, pl.ds(r0, B)]
        yi = ys_ref[0, 0, pl.ds(r0, B)]

        def dist_to(c0):
            xtb = xt_ref[0, 0, pl.ds(c0, CB)]
            ytb = yt_ref[0, 0, pl.ds(c0, CB)]
            dx = xi[:, None] - xtb[None, :]
            dy = yi[:, None] - ytb[None, :]
            return jnp.sqrt(dx * dx + dy * dy)

        def nsum_f(cj, acc):
            c0 = cj * CB
            d = dist_to(c0)
            vtb = vt_ref[0, 0, pl.ds(c0, CB)]
            return acc + jnp.sum(jnp.where(vtb[None, :] > 0, d * d, 0.0), axis=1)

        nsum = lax.fori_loop(0, nbt2, nsum_f, jnp.zeros((B,), jnp.float32))
        nr = jnp.sqrt(nsum)
        nrm_s[0, pl.ds(r0, B)] = nr
        nc = jnp.maximum(nr, 1e-12)

        def store_f(cj, _c):
            c0 = cj * CB
            d = dist_to(c0)
            ebuf[pl.ds(r0, B), pl.ds(c0, CB)] = d / nc[:, None]
            return 0

        lax.fori_loop(0, nbt2, store_f, 0)

        mp = jnp.full((B,), -inf, jnp.float32)
        jp = jnp.full((B,), -1.0, jnp.float32)
        for k in range(K):
            def scan_f(cj, st):
                mm, jj_m = st
                c0 = cj * CB
                C = ebuf[pl.ds(r0, B), pl.ds(c0, CB)]
                jj = lax.broadcasted_iota(jnp.int32, (B, CB), 1).astype(jnp.float32) + (c0).astype(jnp.float32)
                vtb = vt_ref[0, 0, pl.ds(c0, CB)]
                after = (C > mp[:, None]) | ((C == mp[:, None]) & (jj > jp[:, None]))
                cand = jnp.where((vtb[None, :] > 0) & after, C, inf)
                mb = jnp.min(cand, axis=1)
                jb = jnp.min(jnp.where(cand == mb[:, None], jj, BIG), axis=1)
                better = (mb < mm) | ((mb == mm) & (jb < jj_m))
                return (jnp.where(better, mb, mm), jnp.where(better, jb, jj_m))

            mp, jp = lax.fori_loop(0, nbt2, scan_f,
                                   (jnp.full((B,), inf, jnp.float32),
                                    jnp.full((B,), BIG, jnp.float32)))
        rv_s[0, pl.ds(r0, B)] = mp
        rj_s[0, pl.ds(r0, B)] = jp
        return 0

    lax.fori_loop(0, nbs, row_block, 0)

    def col_block(cj, _):
        c0 = cj * CB
        mp = jnp.full((CB,), inf, jnp.float32)
        ip = jnp.full((CB,), -1.0, jnp.float32)
        for k in range(K):
            def scan_f(bi, st):
                mm, ii_m = st
                r0 = bi * B
                C = ebuf[pl.ds(r0, B), pl.ds(c0, CB)]
                ii = lax.broadcasted_iota(jnp.int32, (B, CB), 0).astype(jnp.float32) + (r0).astype(jnp.float32)
                vsb = vs_ref[0, 0, pl.ds(r0, B)]
                after = (C < mp[None, :]) | ((C == mp[None, :]) & (ii > ip[None, :]))
                cand = jnp.where((vsb[:, None] > 0) & after, C, -inf)
                mb = jnp.max(cand, axis=0)
                ib = jnp.min(jnp.where(cand == mb[None, :], ii, BIG), axis=0)
                better = (mb > mm) | ((mb == mm) & (ib < ii_m))
                return (jnp.where(better, mb, mm), jnp.where(better, ib, ii_m))

            mp, ip = lax.fori_loop(0, nbs, scan_f,
                                   (jnp.full((CB,), -inf, jnp.float32),
                                    jnp.full((CB,), BIG, jnp.float32)))
        cv_s[0, pl.ds(c0, CB)] = mp
        ci_s[0, pl.ds(c0, CB)] = ip
        return 0

    lax.fori_loop(0, nbt2, col_block, 0)

    def ebuild_row(bi, _r):
        r0 = bi * B

        def ebuild(cj, _c):
            c0 = cj * CB
            C = ebuf[pl.ds(r0, B), pl.ds(c0, CB)]
            jj = lax.broadcasted_iota(jnp.int32, (B, CB), 1).astype(jnp.float32) + (c0).astype(jnp.float32)
            ii = lax.broadcasted_iota(jnp.int32, (B, CB), 0).astype(jnp.float32) + (r0).astype(jnp.float32)
            vsb = vs_ref[0, 0, pl.ds(r0, B)]
            vtb = vt_ref[0, 0, pl.ds(c0, CB)]
            rv = rv_s[0, pl.ds(r0, B)]
            rj = rj_s[0, pl.ds(r0, B)]
            cv = cv_s[0, pl.ds(c0, CB)]
            ci = ci_s[0, pl.ds(c0, CB)]
            rowsel = (C < rv[:, None]) | ((C == rv[:, None]) & (jj <= rj[:, None]))
            vmask = vsb[:, None] > 0
            cneg = jnp.where(vmask, C, -inf)
            colsel = (cneg > cv[None, :]) | ((cneg == cv[None, :]) & (ii <= ci[None, :]))
            act = (rowsel | colsel) & vmask & (vtb[None, :] > 0)
            ebf[pl.ds(r0, B), pl.ds(c0, CB)] = jnp.where(
                act, jnp.exp(C * (-1.0 / EPS)), 0.0).astype(jnp.bfloat16)
            return 0

        lax.fori_loop(0, nbt2, ebuild, 0)
        return 0

    lax.fori_loop(0, nbs, ebuild_row, 0)

    u_s[0, :] = jnp.zeros((N,), jnp.float32)
    v_s[0, :] = jnp.zeros((N,), jnp.float32)
    av_s[0, :] = jnp.zeros((N,), jnp.float32)
    bv_s[0, :] = jnp.ones((N,), jnp.float32)

    def sink_cond(st):
        it, err = st
        return (it < MAX_ITER) & (err >= THRESH)

    def sink_body(st):
        it, _ = st

        def ublk(bi, errac):
            r0 = bi * B

            def inner(cj, acc):
                c0 = cj * CB
                eb = ebf[pl.ds(r0, B), pl.ds(c0, CB)].astype(jnp.float32)
                bb = bv_s[0, pl.ds(c0, CB)]
                return acc + jnp.sum(eb * bb[None, :], axis=1)

            acc = lax.fori_loop(0, nbt2, inner, jnp.zeros((B,), jnp.float32))
            uold = u_s[0, pl.ds(r0, B)]
            a = jnp.exp(uold * (1.0 / EPS))
            r = a * acc
            unew = EPS * (jnp.log(mu_ref[0, 0, pl.ds(r0, B)] + 1e-8) - jnp.log(r + 1e-8)) + uold
            u_s[0, pl.ds(r0, B)] = unew
            av_s[0, pl.ds(r0, B)] = jnp.exp(unew * (1.0 / EPS))
            return errac + jnp.sum(jnp.abs(unew - uold))

        err2 = lax.fori_loop(0, nbs, ublk, jnp.float32(0.0))

        def vblk(cj, _c):
            c0 = cj * CB

            def inner(bi, acc):
                r0 = bi * B
                eb = ebf[pl.ds(r0, B), pl.ds(c0, CB)].astype(jnp.float32)
                aa = av_s[0, pl.ds(r0, B)]
                return acc + jnp.sum(eb * aa[:, None], axis=0)

            acc = lax.fori_loop(0, nbs, inner, jnp.zeros((CB,), jnp.float32))
            vold = v_s[0, pl.ds(c0, CB)]
            bcur = bv_s[0, pl.ds(c0, CB)]
            r = bcur * acc
            vnew = EPS * (jnp.log(nu_ref[0, 0, pl.ds(c0, CB)] + 1e-8) - jnp.log(r + 1e-8)) + vold
            v_s[0, pl.ds(c0, CB)] = vnew
            bv_s[0, pl.ds(c0, CB)] = jnp.exp(vnew * (1.0 / EPS))
            return 0

        lax.fori_loop(0, nbt2, vblk, 0)
        return (it + 1, err2)

    lax.while_loop(sink_cond, sink_body, (jnp.int32(0), inf))

    def fin_blk(bi, tot):
        r0 = bi * B
        aa = av_s[0, pl.ds(r0, B)]

        def inner(cj, t2):
            c0 = cj * CB
            Cb = ebuf[pl.ds(r0, B), pl.ds(c0, CB)]
            eb = ebf[pl.ds(r0, B), pl.ds(c0, CB)].astype(jnp.float32)
            bb = bv_s[0, pl.ds(c0, CB)]
            pi = eb * aa[:, None] * bb[None, :]
            return t2 + jnp.sum(pi * Cb)

        return tot + lax.fori_loop(0, nbt2, inner, jnp.float32(0.0))

    tot = lax.fori_loop(0, nbs, fin_blk, jnp.float32(0.0))
    out_ref[0, 0] = out_ref[0, 0] + tot


def _gw_call(xs, ys, xt, yt, vsf, vtf, mu, nu, interpret=False):
    batch = xs.shape[0]
    vec = pl.BlockSpec((1, 1, N), lambda bb: (bb, 0, 0))
    return pl.pallas_call(
        _gw_body,
        grid=(batch,),
        in_specs=[vec] * 8,
        out_specs=pl.BlockSpec((1, 1), lambda bb: (0, 0), memory_space=pltpu.SMEM),
        out_shape=jax.ShapeDtypeStruct((1, 1), jnp.float32),
        scratch_shapes=[
            pltpu.VMEM((N, N), jnp.float32),
            pltpu.VMEM((N, N), jnp.bfloat16),
            pltpu.VMEM((1, N), jnp.float32),
            pltpu.VMEM((1, N), jnp.float32),
            pltpu.VMEM((1, N), jnp.float32),
            pltpu.VMEM((1, N), jnp.float32),
            pltpu.VMEM((1, N), jnp.float32),
            pltpu.VMEM((1, N), jnp.float32),
            pltpu.VMEM((1, N), jnp.float32),
            pltpu.VMEM((1, N), jnp.float32),
            pltpu.VMEM((1, N), jnp.float32),
        ],
        compiler_params=pltpu.CompilerParams(
            dimension_semantics=("arbitrary",),
            vmem_limit_bytes=62 * 1024 * 1024,
        ),
        interpret=interpret,
    )(xs, ys, xt, yt, vsf, vtf, mu, nu)


def _coords_compact(proc):
    flat = proc.reshape(-1)
    nz = flat != 0.0
    n = jnp.sum(nz.astype(jnp.int32))
    pos = jnp.arange(flat.shape[0], dtype=jnp.int32)
    cum = jnp.cumsum(nz.astype(jnp.int32)) - 1
    scat = jnp.where(nz, cum, MAX_NUM)
    order_rm = jnp.zeros((MAX_NUM,), jnp.int32).at[scat].set(pos, mode="drop")

    def by_value(_):
        _, bv = jax.lax.top_k(jnp.where(nz, flat, -jnp.inf), MAX_NUM)
        return bv.astype(jnp.int32)

    order = jax.lax.cond(n > MAX_NUM, by_value, lambda _: order_rm, None)
    valid = jnp.arange(MAX_NUM) < jnp.clip(n, 1, MAX_NUM)
    order = jnp.where(valid, order, 0)
    return order // W, order % W, valid


def _marginal(proc, r, c, valid):
    mask = (proc > 0.5).astype(jnp.float32)
    kern = jnp.ones((1, 1, 3, 3), jnp.float32)
    deg = jax.lax.conv_general_dilated(mask[None, None], kern, (1, 1), "SAME")[0, 0] * mask
    deg = deg / jnp.sum(deg)
    mu = deg[r, c] * proc[r, c]
    return jnp.where(valid, mu, 0.0)


def _pad_f32(x):
    return jnp.pad(x.astype(jnp.float32), (0, N - MAX_NUM))


def kernel(pred, target, interpret=False):
    batch = pred.shape[0]
    cols = {k: [] for k in ("xs", "ys", "xt", "yt", "vs", "vt", "mu", "nu")}
    for i in range(batch):
        s = jax.nn.sigmoid(pred[i, 0])
        pm = s * (s > 0.5).astype(jnp.float32)
        tr = target[i, 0]
        tm = tr * (tr > 0.5).astype(jnp.float32)
        rs, cs, vsb = _coords_compact(pm)
        rt, ct, vtb = _coords_compact(tm)
        cols["xs"].append(_pad_f32(rs))
        cols["ys"].append(_pad_f32(cs))
        cols["xt"].append(_pad_f32(rt))
        cols["yt"].append(_pad_f32(ct))
        cols["vs"].append(_pad_f32(vsb))
        cols["vt"].append(_pad_f32(vtb))
        cols["mu"].append(_pad_f32(_marginal(pm, rs, cs, vsb)))
        cols["nu"].append(_pad_f32(_marginal(tm, rt, ct, vtb)))
    args = [jnp.stack(cols[k])[:, None, :] for k in ("xs", "ys", "xt", "yt", "vs", "vt", "mu", "nu")]
    out = _gw_call(*args, interpret=interpret)
    return out[0, 0] / batch


# speedup vs baseline: 1.0055x; 1.0055x over previous
import jax
import jax.numpy as jnp
from jax import lax
from jax.experimental import pallas as pl
from jax.experimental.pallas import tpu as pltpu

EPS = 0.1
THRESH = 0.1
MAX_ITER = 30
MAX_NUM = 3000
K = 8
W = 224

N = 3072
B = 256
CB = 512
NRB = N // B
NCB = N // CB
BIG = 1e9


def _gw_body(xs_ref, ys_ref, xt_ref, yt_ref, vs_ref, vt_ref, mu_ref, nu_ref,
             out_ref, ebuf, nrm_s, rv_s, rj_s, cv_s, ci_s, u_s, v_s, av_s, bv_s):
    b = pl.program_id(0)

    @pl.when(b == 0)
    def _():
        out_ref[0, 0] = 0.0

    vs = vs_ref[0, 0, :]
    vt = vt_ref[0, 0, :]
    nsi = jnp.sum(vs).astype(jnp.int32)
    nti = jnp.sum(vt).astype(jnp.int32)
    nbs = jnp.maximum((nsi + B - 1) // B, 1)
    nbt2 = jnp.maximum((nti + CB - 1) // CB, 1)
    inf = jnp.float32(jnp.inf)

    def row_block(bi, _):
        r0 = bi * B
        xi = xs_ref[0, 0, pl.ds(r0, B)]
        yi = ys_ref[0, 0, pl.ds(r0, B)]

        def dist_to(c0):
            xtb = xt_ref[0, 0, pl.ds(c0, CB)]
            ytb = yt_ref[0, 0, pl.ds(c0, CB)]
            dx = xi[:, None] - xtb[None, :]
            dy = yi[:, None] - ytb[None, :]
            return jnp.sqrt(dx * dx + dy * dy)

        def nsum_f(cj, acc):
            c0 = cj * CB
            d = dist_to(c0)
            vtb = vt_ref[0, 0, pl.ds(c0, CB)]
            return acc + jnp.sum(jnp.where(vtb[None, :] > 0, d * d, 0.0), axis=1)

        nsum = lax.fori_loop(0, nbt2, nsum_f, jnp.zeros((B,), jnp.float32))
        nr = jnp.sqrt(nsum)
        nrm_s[0, pl.ds(r0, B)] = nr
        nc = jnp.maximum(nr, 1e-12)

        def store_f(cj, _c):
            c0 = cj * CB
            d = dist_to(c0)
            ebuf[pl.ds(r0, B), pl.ds(c0, CB)] = d / nc[:, None]
            return 0

        lax.fori_loop(0, nbt2, store_f, 0)

        mp = jnp.full((B,), -inf, jnp.float32)
        jp = jnp.full((B,), -1.0, jnp.float32)
        for k in range(K):
            def scan_f(cj, st):
                mm, jj_m = st
                c0 = cj * CB
                C = ebuf[pl.ds(r0, B), pl.ds(c0, CB)]
                jj = lax.broadcasted_iota(jnp.int32, (B, CB), 1).astype(jnp.float32) + (c0).astype(jnp.float32)
                vtb = vt_ref[0, 0, pl.ds(c0, CB)]
                after = (C > mp[:, None]) | ((C == mp[:, None]) & (jj > jp[:, None]))
                cand = jnp.where((vtb[None, :] > 0) & after, C, inf)
                mb = jnp.min(cand, axis=1)
                jb = jnp.min(jnp.where(cand == mb[:, None], jj, BIG), axis=1)
                better = (mb < mm) | ((mb == mm) & (jb < jj_m))
                return (jnp.where(better, mb, mm), jnp.where(better, jb, jj_m))

            mp, jp = lax.fori_loop(0, nbt2, scan_f,
                                   (jnp.full((B,), inf, jnp.float32),
                                    jnp.full((B,), BIG, jnp.float32)))
        rv_s[0, pl.ds(r0, B)] = mp
        rj_s[0, pl.ds(r0, B)] = jp
        return 0

    lax.fori_loop(0, nbs, row_block, 0)

    def col_block(cj, _):
        c0 = cj * CB
        mp = jnp.full((CB,), inf, jnp.float32)
        ip = jnp.full((CB,), -1.0, jnp.float32)
        for k in range(K):
            def scan_f(bi, st):
                mm, ii_m = st
                r0 = bi * B
                C = ebuf[pl.ds(r0, B), pl.ds(c0, CB)]
                ii = lax.broadcasted_iota(jnp.int32, (B, CB), 0).astype(jnp.float32) + (r0).astype(jnp.float32)
                vsb = vs_ref[0, 0, pl.ds(r0, B)]
                after = (C < mp[None, :]) | ((C == mp[None, :]) & (ii > ip[None, :]))
                cand = jnp.where((vsb[:, None] > 0) & after, C, -inf)
                mb = jnp.max(cand, axis=0)
                ib = jnp.min(jnp.where(cand == mb[None, :], ii, BIG), axis=0)
                better = (mb > mm) | ((mb == mm) & (ib < ii_m))
                return (jnp.where(better, mb, mm), jnp.where(better, ib, ii_m))

            mp, ip = lax.fori_loop(0, nbs, scan_f,
                                   (jnp.full((CB,), -inf, jnp.float32),
                                    jnp.full((CB,), BIG, jnp.float32)))
        cv_s[0, pl.ds(c0, CB)] = mp
        ci_s[0, pl.ds(c0, CB)] = ip
        return 0

    lax.fori_loop(0, nbt2, col_block, 0)

    def ebuild_row(bi, _r):
        r0 = bi * B

        def ebuild(cj, _c):
            c0 = cj * CB
            C = ebuf[pl.ds(r0, B), pl.ds(c0, CB)]
            jj = lax.broadcasted_iota(jnp.int32, (B, CB), 1).astype(jnp.float32) + (c0).astype(jnp.float32)
            ii = lax.broadcasted_iota(jnp.int32, (B, CB), 0).astype(jnp.float32) + (r0).astype(jnp.float32)
            vsb = vs_ref[0, 0, pl.ds(r0, B)]
            vtb = vt_ref[0, 0, pl.ds(c0, CB)]
            rv = rv_s[0, pl.ds(r0, B)]
            rj = rj_s[0, pl.ds(r0, B)]
            cv = cv_s[0, pl.ds(c0, CB)]
            ci = ci_s[0, pl.ds(c0, CB)]
            rowsel = (C < rv[:, None]) | ((C == rv[:, None]) & (jj <= rj[:, None]))
            vmask = vsb[:, None] > 0
            cneg = jnp.where(vmask, C, -inf)
            colsel = (cneg > cv[None, :]) | ((cneg == cv[None, :]) & (ii <= ci[None, :]))
            act = (rowsel | colsel) & vmask & (vtb[None, :] > 0)
            ebuf[pl.ds(r0, B), pl.ds(c0, CB)] = jnp.where(act, jnp.exp(C * (-1.0 / EPS)), 0.0)
            return 0

        lax.fori_loop(0, nbt2, ebuild, 0)
        return 0

    lax.fori_loop(0, nbs, ebuild_row, 0)

    u_s[0, :] = jnp.zeros((N,), jnp.float32)
    v_s[0, :] = jnp.zeros((N,), jnp.float32)
    av_s[0, :] = jnp.zeros((N,), jnp.float32)
    bv_s[0, :] = jnp.ones((N,), jnp.float32)

    def sink_cond(st):
        it, err = st
        return (it < MAX_ITER) & (err >= THRESH)

    def sink_body(st):
        it, _ = st

        def ublk(bi, errac):
            r0 = bi * B

            def inner(cj, acc):
                c0 = cj * CB
                eb = ebuf[pl.ds(r0, B), pl.ds(c0, CB)]
                bb = bv_s[0, pl.ds(c0, CB)]
                return acc + jnp.sum(eb * bb[None, :], axis=1)

            acc = lax.fori_loop(0, nbt2, inner, jnp.zeros((B,), jnp.float32))
            uold = u_s[0, pl.ds(r0, B)]
            a = jnp.exp(uold * (1.0 / EPS))
            r = a * acc
            unew = EPS * (jnp.log(mu_ref[0, 0, pl.ds(r0, B)] + 1e-8) - jnp.log(r + 1e-8)) + uold
            u_s[0, pl.ds(r0, B)] = unew
            av_s[0, pl.ds(r0, B)] = jnp.exp(unew * (1.0 / EPS))
            return errac + jnp.sum(jnp.abs(unew - uold))

        err2 = lax.fori_loop(0, nbs, ublk, jnp.float32(0.0))

        def vblk(cj, _c):
            c0 = cj * CB

            def inner(bi, acc):
                r0 = bi * B
                eb = ebuf[pl.ds(r0, B), pl.ds(c0, CB)]
                aa = av_s[0, pl.ds(r0, B)]
                return acc + jnp.sum(eb * aa[:, None], axis=0)

            acc = lax.fori_loop(0, nbs, inner, jnp.zeros((CB,), jnp.float32))
            vold = v_s[0, pl.ds(c0, CB)]
            bcur = bv_s[0, pl.ds(c0, CB)]
            r = bcur * acc
            vnew = EPS * (jnp.log(nu_ref[0, 0, pl.ds(c0, CB)] + 1e-8) - jnp.log(r + 1e-8)) + vold
            v_s[0, pl.ds(c0, CB)] = vnew
            bv_s[0, pl.ds(c0, CB)] = jnp.exp(vnew * (1.0 / EPS))
            return 0

        lax.fori_loop(0, nbt2, vblk, 0)
        return (it + 1, err2)

    lax.while_loop(sink_cond, sink_body, (jnp.int32(0), inf))

    def fin_blk(bi, tot):
        r0 = bi * B
        xi = xs_ref[0, 0, pl.ds(r0, B)]
        yi = ys_ref[0, 0, pl.ds(r0, B)]
        nc = jnp.maximum(nrm_s[0, pl.ds(r0, B)], 1e-12)
        aa = av_s[0, pl.ds(r0, B)]

        def inner(cj, t2):
            c0 = cj * CB
            xtb = xt_ref[0, 0, pl.ds(c0, CB)]
            ytb = yt_ref[0, 0, pl.ds(c0, CB)]
            dx = xi[:, None] - xtb[None, :]
            dy = yi[:, None] - ytb[None, :]
            d = jnp.sqrt(dx * dx + dy * dy)
            C = d / nc[:, None]
            eb = ebuf[pl.ds(r0, B), pl.ds(c0, CB)]
            bb = bv_s[0, pl.ds(c0, CB)]
            pi = eb * aa[:, None] * bb[None, :]
            return t2 + jnp.sum(pi * C)

        return tot + lax.fori_loop(0, nbt2, inner, jnp.float32(0.0))

    tot = lax.fori_loop(0, nbs, fin_blk, jnp.float32(0.0))
    out_ref[0, 0] = out_ref[0, 0] + tot


def _gw_call(xs, ys, xt, yt, vsf, vtf, mu, nu, interpret=False):
    batch = xs.shape[0]
    vec = pl.BlockSpec((1, 1, N), lambda bb: (bb, 0, 0))
    return pl.pallas_call(
        _gw_body,
        grid=(batch,),
        in_specs=[vec] * 8,
        out_specs=pl.BlockSpec((1, 1), lambda bb: (0, 0), memory_space=pltpu.SMEM),
        out_shape=jax.ShapeDtypeStruct((1, 1), jnp.float32),
        scratch_shapes=[
            pltpu.VMEM((N, N), jnp.float32),
            pltpu.VMEM((1, N), jnp.float32),
            pltpu.VMEM((1, N), jnp.float32),
            pltpu.VMEM((1, N), jnp.float32),
            pltpu.VMEM((1, N), jnp.float32),
            pltpu.VMEM((1, N), jnp.float32),
            pltpu.VMEM((1, N), jnp.float32),
            pltpu.VMEM((1, N), jnp.float32),
            pltpu.VMEM((1, N), jnp.float32),
            pltpu.VMEM((1, N), jnp.float32),
        ],
        compiler_params=pltpu.CompilerParams(
            dimension_semantics=("arbitrary",),
            vmem_limit_bytes=62 * 1024 * 1024,
        ),
        interpret=interpret,
    )(xs, ys, xt, yt, vsf, vtf, mu, nu)


def _coords_compact(proc):
    flat = proc.reshape(-1)
    nz = flat != 0.0
    n = jnp.sum(nz.astype(jnp.int32))
    pos = jnp.arange(flat.shape[0], dtype=jnp.int32)
    cum = jnp.cumsum(nz.astype(jnp.int32)) - 1
    scat = jnp.where(nz, cum, MAX_NUM)
    order_rm = jnp.zeros((MAX_NUM,), jnp.int32).at[scat].set(pos, mode="drop")

    def by_value(_):
        _, bv = jax.lax.top_k(jnp.where(nz, flat, -jnp.inf), MAX_NUM)
        return bv.astype(jnp.int32)

    order = jax.lax.cond(n > MAX_NUM, by_value, lambda _: order_rm, None)
    valid = jnp.arange(MAX_NUM) < jnp.clip(n, 1, MAX_NUM)
    order = jnp.where(valid, order, 0)
    return order // W, order % W, valid


def _marginal(proc, r, c, valid):
    mask = (proc > 0.5).astype(jnp.float32)
    kern = jnp.ones((1, 1, 3, 3), jnp.float32)
    deg = jax.lax.conv_general_dilated(mask[None, None], kern, (1, 1), "SAME")[0, 0] * mask
    deg = deg / jnp.sum(deg)
    mu = deg[r, c] * proc[r, c]
    return jnp.where(valid, mu, 0.0)


def _pad_f32(x):
    return jnp.pad(x.astype(jnp.float32), (0, N - MAX_NUM))


def kernel(pred, target, interpret=False):
    batch = pred.shape[0]
    cols = {k: [] for k in ("xs", "ys", "xt", "yt", "vs", "vt", "mu", "nu")}
    for i in range(batch):
        s = jax.nn.sigmoid(pred[i, 0])
        pm = s * (s > 0.5).astype(jnp.float32)
        tr = target[i, 0]
        tm = tr * (tr > 0.5).astype(jnp.float32)
        rs, cs, vsb = _coords_compact(pm)
        rt, ct, vtb = _coords_compact(tm)
        cols["xs"].append(_pad_f32(rs))
        cols["ys"].append(_pad_f32(cs))
        cols["xt"].append(_pad_f32(rt))
        cols["yt"].append(_pad_f32(ct))
        cols["vs"].append(_pad_f32(vsb))
        cols["vt"].append(_pad_f32(vtb))
        cols["mu"].append(_pad_f32(_marginal(pm, rs, cs, vsb)))
        cols["nu"].append(_pad_f32(_marginal(tm, rt, ct, vtb)))
    args = [jnp.stack(cols[k])[:, None, :] for k in ("xs", "ys", "xt", "yt", "vs", "vt", "mu", "nu")]
    out = _gw_call(*args, interpret=interpret)
    return out[0, 0] / batch
